# Initial kernel scaffold; baseline (speedup 1.0000x reference)
import jax
import jax.numpy as jnp
from jax.experimental import pallas as pl


def kernel(positions, neighbors, neighbor_mask, cell, cell_offsets):
    raise NotImplementedError("write your pallas kernel here")


# speedup vs baseline: 20.1539x; 20.1539x over previous
import functools

import jax
import jax.numpy as jnp
from jax import lax
from jax.experimental import pallas as pl
from jax.experimental.pallas import tpu as pltpu
from jax.experimental.pallas import tpu_sc as plsc

NC = 2
NS = 16
NW = NC * NS
LANES = 16


def _sqrt16(x):
    i = lax.bitcast_convert_type(x, jnp.int32)
    y = lax.bitcast_convert_type(
        (i >> 1) + jnp.int32(0x1FBD1DF5), jnp.float32)
    y = 0.5 * (y + x / y)
    y = 0.5 * (y + x / y)
    return y


def _distances_body(B, N, Nbh, chunk_atoms,
                    pos_hbm, nbr_hbm, mask_hbm, cell_hbm, off_hbm, out_hbm,
                    table_v, cell_v, nbr_v, mask_v, off_v, out_v):
    wpb = NW // B
    atoms_per_worker = N // wpb
    nchunks = atoms_per_worker // chunk_atoms
    chunk_e = chunk_atoms * Nbh
    vregs_per_atom = Nbh // LANES

    wid = lax.axis_index("s") * NC + lax.axis_index("c")
    b = wid // wpb
    sub = wid % wpb

    pltpu.sync_copy(pos_hbm.at[b], table_v)
    pltpu.sync_copy(cell_hbm.at[b], cell_v)

    cell_vec = cell_v[pl.ds(0, LANES)]

    def _bc(i):
        return jnp.full((LANES,), cell_vec[i], dtype=jnp.float32)

    c00, c01, c02 = _bc(0), _bc(1), _bc(2)
    c10, c11, c12 = _bc(3), _bc(4), _bc(5)
    c20, c21, c22 = _bc(6), _bc(7), _bc(8)

    iota3 = lax.iota(jnp.int32, LANES) * 3
    a_base = sub * atoms_per_worker

    def chunk_body(ci, carry):
        a0 = a_base + ci * chunk_atoms
        e0 = pl.multiple_of(a0 * Nbh, 8)
        pltpu.sync_copy(nbr_hbm.at[b, pl.ds(e0, chunk_e)], nbr_v)
        pltpu.sync_copy(mask_hbm.at[b, pl.ds(e0, chunk_e)], mask_v)
        pltpu.sync_copy(off_hbm.at[b, pl.ds(e0 * 3, chunk_e * 3)], off_v)

        def vreg_body(v, carry2):
            le = v * LANES
            a3v = jnp.full((LANES,), (a0 + v // vregs_per_atom) * 3,
                           dtype=jnp.int32)
            nb3 = nbr_v[pl.ds(le, LANES)] * 3
            gx = plsc.load_gather(table_v, [nb3])
            gy = plsc.load_gather(table_v, [nb3 + 1])
            gz = plsc.load_gather(table_v, [nb3 + 2])
            cx = plsc.load_gather(table_v, [a3v])
            cy = plsc.load_gather(table_v, [a3v + 1])
            cz = plsc.load_gather(table_v, [a3v + 2])
            oidx = iota3 + le * 3
            ox = plsc.load_gather(off_v, [oidx])
            oy = plsc.load_gather(off_v, [oidx + 1])
            oz = plsc.load_gather(off_v, [oidx + 2])
            dx = gx - cx + (ox * c00 + oy * c10 + oz * c20)
            dy = gy - cy + (ox * c01 + oy * c11 + oz * c21)
            dz = gz - cz + (ox * c02 + oy * c12 + oz * c22)
            d2 = dx * dx + dy * dy + dz * dz
            dist = _sqrt16(d2)
            m = mask_v[pl.ds(le, LANES)]
            out_v[pl.ds(le, LANES)] = jnp.where(m != 0.0, dist, 0.0)
            return carry2

        lax.fori_loop(0, chunk_e // LANES, vreg_body, 0, unroll=False)
        pltpu.sync_copy(out_v, out_hbm.at[b, pl.ds(e0, chunk_e)])
        return carry

    lax.fori_loop(0, nchunks, chunk_body, 0, unroll=False)


def kernel(positions, neighbors, neighbor_mask, cell, cell_offsets):
    B, N, Nbh = neighbors.shape
    chunk_atoms = 100
    chunk_e = chunk_atoms * Nbh

    pos_flat = positions.reshape(B, N * 3)
    nbr_flat = neighbors.astype(jnp.int32).reshape(B, N * Nbh)
    mask_flat = neighbor_mask.reshape(B, N * Nbh)
    cell_pad = jnp.pad(cell.reshape(B, 9), ((0, 0), (0, 7)))
    off_flat = cell_offsets.reshape(B, N * Nbh * 3)

    body = functools.partial(_distances_body, B, N, Nbh, chunk_atoms)
    out = pl.kernel(
        body,
        out_type=jax.ShapeDtypeStruct((B, N * Nbh), jnp.float32),
        mesh=plsc.VectorSubcoreMesh(core_axis_name="c", subcore_axis_name="s"),
        compiler_params=pltpu.CompilerParams(needs_layout_passes=False),
        scratch_types=[
            pltpu.VMEM((N * 3,), jnp.float32),
            pltpu.VMEM((16,), jnp.float32),
            pltpu.VMEM((chunk_e,), jnp.int32),
            pltpu.VMEM((chunk_e,), jnp.float32),
            pltpu.VMEM((chunk_e * 3,), jnp.float32),
            pltpu.VMEM((chunk_e,), jnp.float32),
        ],
    )(pos_flat, nbr_flat, mask_flat, cell_pad, off_flat)
    return out.reshape(B, N, Nbh)


# speedup vs baseline: 20.9209x; 1.0381x over previous
import functools

import jax
import jax.numpy as jnp
from jax import lax
from jax.experimental import pallas as pl
from jax.experimental.pallas import tpu as pltpu
from jax.experimental.pallas import tpu_sc as plsc

NC = 2
NS = 16
NW = NC * NS
LANES = 16


def _dist_from_sq(d2):
    i = lax.bitcast_convert_type(d2, jnp.int32)
    y = lax.bitcast_convert_type(jnp.int32(0x5F3759DF) - (i >> 1),
                                 jnp.float32)
    xh = 0.5 * d2
    y = y * (1.5 - xh * y * y)
    y = y * (1.5 - xh * y * y)
    return d2 * y


def _distances_body(B, N, Nbh, chunk_atoms,
                    pos_hbm, nbr_hbm, mask_hbm, cell_hbm, off_hbm, out_hbm,
                    table_v, cell_v, nbr_v0, nbr_v1, mask_v0, mask_v1,
                    off_v0, off_v1, out_v0, out_v1, sem_a, sem_b):
    slot_refs = ((nbr_v0, mask_v0, off_v0, out_v0),
                 (nbr_v1, mask_v1, off_v1, out_v1))
    wpb = NW // B
    atoms_per_worker = N // wpb
    nchunks = atoms_per_worker // chunk_atoms
    chunk_e = chunk_atoms * Nbh
    upa = Nbh // LANES

    wid = lax.axis_index("s") * NC + lax.axis_index("c")
    b = wid // wpb
    sub = wid % wpb

    pltpu.sync_copy(pos_hbm.at[b], table_v)
    pltpu.sync_copy(cell_hbm.at[b], cell_v)

    cell_vec = cell_v[pl.ds(0, LANES)]

    def _bc(i):
        return jnp.full((LANES,), cell_vec[i], dtype=jnp.float32)

    c00, c01, c02 = _bc(0), _bc(1), _bc(2)
    c10, c11, c12 = _bc(3), _bc(4), _bc(5)
    c20, c21, c22 = _bc(6), _bc(7), _bc(8)

    iota3 = lax.iota(jnp.int32, LANES) * 3
    a_base = sub * atoms_per_worker

    def _in_copies(ci, slot, sem):
        nbr_r, mask_r, off_r, _ = slot_refs[slot]
        a0 = a_base + ci * chunk_atoms
        e0 = pl.multiple_of(a0 * Nbh, 8)
        return (
            pltpu.make_async_copy(
                nbr_hbm.at[b, pl.ds(e0, chunk_e)], nbr_r, sem),
            pltpu.make_async_copy(
                mask_hbm.at[b, pl.ds(e0, chunk_e)], mask_r, sem),
            pltpu.make_async_copy(
                off_hbm.at[b, pl.ds(e0 * 3, chunk_e * 3)], off_r, sem),
        )

    def _start(ci, slot, sem):
        for cp in _in_copies(ci, slot, sem):
            cp.start()

    def _wait(ci, slot, sem):
        for cp in _in_copies(ci, slot, sem):
            cp.wait()

    def _compute_chunk(ci, slot):
        a0 = a_base + ci * chunk_atoms
        nbr_r, mask_r, off_r, out_r = slot_refs[slot]

        def atom_body(ai, carry):
            a3v = jnp.full((LANES,), (a0 + ai) * 3, dtype=jnp.int32)
            cx = plsc.load_gather(table_v, [a3v])
            cy = plsc.load_gather(table_v, [a3v + 1])
            cz = plsc.load_gather(table_v, [a3v + 2])
            le0 = ai * Nbh
            for u in range(upa):
                le = le0 + u * LANES
                nb3 = nbr_r[pl.ds(le, LANES)] * 3
                gx = plsc.load_gather(table_v, [nb3])
                gy = plsc.load_gather(table_v, [nb3 + 1])
                gz = plsc.load_gather(table_v, [nb3 + 2])
                oidx = iota3 + le * 3
                ox = plsc.load_gather(off_r, [oidx])
                oy = plsc.load_gather(off_r, [oidx + 1])
                oz = plsc.load_gather(off_r, [oidx + 2])
                dx = gx - cx + (ox * c00 + oy * c10 + oz * c20)
                dy = gy - cy + (ox * c01 + oy * c11 + oz * c21)
                dz = gz - cz + (ox * c02 + oy * c12 + oz * c22)
                dist = _dist_from_sq(dx * dx + dy * dy + dz * dz)
                m = mask_r[pl.ds(le, LANES)]
                out_r[pl.ds(le, LANES)] = jnp.where(m != 0.0, dist, 0.0)
            return carry

        lax.fori_loop(0, chunk_atoms, atom_body, 0, unroll=False)
        e0 = pl.multiple_of(a0 * Nbh, 8)
        pltpu.sync_copy(out_r, out_hbm.at[b, pl.ds(e0, chunk_e)])

    _start(0, 0, sem_a)

    def pair_body(pi, carry):
        ci0 = pi * 2
        _wait(ci0, 0, sem_a)
        _start(ci0 + 1, 1, sem_b)
        _compute_chunk(ci0, 0)
        _wait(ci0 + 1, 1, sem_b)
        _start(ci0 + 2, 0, sem_a)
        _compute_chunk(ci0 + 1, 1)
        return carry

    lax.fori_loop(0, (nchunks - 1) // 2, pair_body, 0, unroll=False)
    _wait(nchunks - 1, 0, sem_a)
    _compute_chunk(nchunks - 1, 0)


def kernel(positions, neighbors, neighbor_mask, cell, cell_offsets):
    B, N, Nbh = neighbors.shape
    chunk_atoms = 100
    chunk_e = chunk_atoms * Nbh

    pos_flat = positions.reshape(B, N * 3)
    nbr_flat = neighbors.astype(jnp.int32).reshape(B, N * Nbh)
    mask_flat = neighbor_mask.reshape(B, N * Nbh)
    cell_pad = jnp.pad(cell.reshape(B, 9), ((0, 0), (0, 7)))
    off_flat = cell_offsets.reshape(B, N * Nbh * 3)

    body = functools.partial(_distances_body, B, N, Nbh, chunk_atoms)
    out = pl.kernel(
        body,
        out_type=jax.ShapeDtypeStruct((B, N * Nbh), jnp.float32),
        mesh=plsc.VectorSubcoreMesh(core_axis_name="c", subcore_axis_name="s"),
        compiler_params=pltpu.CompilerParams(needs_layout_passes=False),
        scratch_types=[
            pltpu.VMEM((N * 3,), jnp.float32),
            pltpu.VMEM((16,), jnp.float32),
            pltpu.VMEM((chunk_e,), jnp.int32),
            pltpu.VMEM((chunk_e,), jnp.int32),
            pltpu.VMEM((chunk_e,), jnp.float32),
            pltpu.VMEM((chunk_e,), jnp.float32),
            pltpu.VMEM((chunk_e * 3,), jnp.float32),
            pltpu.VMEM((chunk_e * 3,), jnp.float32),
            pltpu.VMEM((chunk_e,), jnp.float32),
            pltpu.VMEM((chunk_e,), jnp.float32),
            pltpu.SemaphoreType.DMA,
            pltpu.SemaphoreType.DMA,
        ],
    )(pos_flat, nbr_flat, mask_flat, cell_pad, off_flat)
    return out.reshape(B, N, Nbh)


# speedup vs baseline: 243.8042x; 11.6536x over previous
import functools

import jax
import jax.numpy as jnp
from jax import lax
from jax.experimental import pallas as pl
from jax.experimental.pallas import tpu as pltpu
from jax.experimental.pallas import tpu_sc as plsc

NC = 2
NS = 16
NW = NC * NS
LANES = 16


def _dist_from_sq(d2):
    i = lax.bitcast_convert_type(d2, jnp.int32)
    y = lax.bitcast_convert_type(jnp.int32(0x5F3759DF) - (i >> 1),
                                 jnp.float32)
    xh = 0.5 * d2
    y = y * (1.5 - xh * y * y)
    y = y * (1.5 - xh * y * y)
    return d2 * y


def _distances_body(B, N, K, W,
                    pos_hbm, nbr_hbm, mask_hbm, cell_hbm, off_hbm, out_hbm,
                    tab_x, tab_y, tab_z, cell_v,
                    nbr_b0, nbr_b1, mask_b0, mask_b1,
                    offx_b0, offx_b1, offy_b0, offy_b1, offz_b0, offz_b1,
                    out_b0, out_b1,
                    sem_a, sem_b, sem_oa, sem_ob):
    wpb = NW // B
    kpw = K // wpb
    ngrp = kpw // 8
    n_pad = -(-N // 128) * 128
    nwin = -(-(n_pad - W) // W) + 1
    last_n0 = n_pad - W
    nchunks = ngrp * nwin
    in_slots = ((nbr_b0, mask_b0, offx_b0, offy_b0, offz_b0),
                (nbr_b1, mask_b1, offx_b1, offy_b1, offz_b1))
    out_slots = (out_b0, out_b1)
    out_sems = (sem_oa, sem_ob)

    wid = lax.axis_index("s") * NC + lax.axis_index("c")
    b = wid // wpb
    k0 = (wid % wpb) * kpw

    pltpu.sync_copy(pos_hbm.at[pl.ds((0 * B + b) * N, N)],
                    tab_x.at[pl.ds(0, N)])
    pltpu.sync_copy(pos_hbm.at[pl.ds((1 * B + b) * N, N)],
                    tab_y.at[pl.ds(0, N)])
    pltpu.sync_copy(pos_hbm.at[pl.ds((2 * B + b) * N, N)],
                    tab_z.at[pl.ds(0, N)])
    pltpu.sync_copy(cell_hbm.at[pl.ds(b * 16, 16)], cell_v)

    cell_vec = cell_v[pl.ds(0, LANES)]

    def _bc(i):
        return jnp.full((LANES,), cell_vec[i], dtype=jnp.float32)

    c00, c01, c02 = _bc(0), _bc(1), _bc(2)
    c10, c11, c12 = _bc(3), _bc(4), _bc(5)
    c20, c21, c22 = _bc(6), _bc(7), _bc(8)

    def _gw(t):
        g = t // nwin
        w = t - g * nwin
        n0 = jnp.minimum(w * W, last_n0)
        return g, pl.multiple_of(n0, 128)

    def _in_copies(t, slot, sem):
        nbr_r, mask_r, ox_r, oy_r, oz_r = in_slots[slot]
        g, n0 = _gw(t)
        rk = b * K + k0 + g * 8
        ro = b * 3 * K + k0 + g * 8
        return (
            pltpu.make_async_copy(
                nbr_hbm.at[pl.ds(rk, 8), pl.ds(n0, W)], nbr_r, sem),
            pltpu.make_async_copy(
                mask_hbm.at[pl.ds(rk, 8), pl.ds(n0, W)], mask_r, sem),
            pltpu.make_async_copy(
                off_hbm.at[pl.ds(ro, 8), pl.ds(n0, W)], ox_r, sem),
            pltpu.make_async_copy(
                off_hbm.at[pl.ds(ro + K, 8), pl.ds(n0, W)], oy_r, sem),
            pltpu.make_async_copy(
                off_hbm.at[pl.ds(ro + 2 * K, 8), pl.ds(n0, W)], oz_r, sem),
        )

    def _start(t, slot, sem):
        for cp in _in_copies(t, slot, sem):
            cp.start()

    def _wait(t, slot, sem):
        for cp in _in_copies(t, slot, sem):
            cp.wait()

    def _out_copy(t, slot):
        g, n0 = _gw(t)
        rk = b * K + k0 + g * 8
        return pltpu.make_async_copy(
            out_slots[slot], out_hbm.at[pl.ds(rk, 8), pl.ds(n0, W)],
            out_sems[slot])

    def _compute(t, slot):
        nbr_r, mask_r, ox_r, oy_r, oz_r = in_slots[slot]
        out_r = out_slots[slot]
        _, n0 = _gw(t)

        def vreg_body(v, carry):
            le = v * LANES
            cx = tab_x[pl.ds(n0 + le, LANES)]
            cy = tab_y[pl.ds(n0 + le, LANES)]
            cz = tab_z[pl.ds(n0 + le, LANES)]
            for s in range(8):
                nb = jnp.clip(nbr_r[s, pl.ds(le, LANES)], 0, N - 1)
                gx = plsc.load_gather(tab_x, [nb])
                gy = plsc.load_gather(tab_y, [nb])
                gz = plsc.load_gather(tab_z, [nb])
                ox = ox_r[s, pl.ds(le, LANES)]
                oy = oy_r[s, pl.ds(le, LANES)]
                oz = oz_r[s, pl.ds(le, LANES)]
                dx = gx - cx + (ox * c00 + oy * c10 + oz * c20)
                dy = gy - cy + (ox * c01 + oy * c11 + oz * c21)
                dz = gz - cz + (ox * c02 + oy * c12 + oz * c22)
                dist = _dist_from_sq(dx * dx + dy * dy + dz * dz)
                m = mask_r[s, pl.ds(le, LANES)]
                out_r[s, pl.ds(le, LANES)] = jnp.where(m != 0.0, dist, 0.0)
            return carry

        lax.fori_loop(0, W // LANES, vreg_body, 0, unroll=False)

    _start(0, 0, sem_a)

    def pair_body(pi, carry):
        t0 = pi * 2
        _wait(t0, 0, sem_a)
        _start(t0 + 1, 1, sem_b)

        @pl.when(pi > 0)
        def _():
            _out_copy(t0 - 2, 0).wait()

        _compute(t0, 0)
        _out_copy(t0, 0).start()

        _wait(t0 + 1, 1, sem_b)

        @pl.when(t0 + 2 < nchunks)
        def _():
            _start(t0 + 2, 0, sem_a)

        @pl.when(pi > 0)
        def _():
            _out_copy(t0 - 1, 1).wait()

        _compute(t0 + 1, 1)
        _out_copy(t0 + 1, 1).start()
        return carry

    lax.fori_loop(0, nchunks // 2, pair_body, 0, unroll=False)
    _out_copy(nchunks - 2, 0).wait()
    _out_copy(nchunks - 1, 1).wait()


def kernel(positions, neighbors, neighbor_mask, cell, cell_offsets):
    B, N, K = neighbors.shape
    W = 512

    pos_1d = jnp.transpose(positions, (2, 0, 1)).reshape(3 * B * N)
    nbr_t = jnp.transpose(neighbors.astype(jnp.int32),
                          (0, 2, 1)).reshape(B * K, N)
    mask_t = jnp.transpose(neighbor_mask, (0, 2, 1)).reshape(B * K, N)
    off_t = jnp.transpose(cell_offsets, (0, 3, 2, 1)).reshape(B * 3 * K, N)
    cell_1d = jnp.pad(cell.reshape(B, 9), ((0, 0), (0, 7))).reshape(B * 16)

    body = functools.partial(_distances_body, B, N, K, W)
    out = pl.kernel(
        body,
        out_type=jax.ShapeDtypeStruct((B * K, N), jnp.float32),
        mesh=plsc.VectorSubcoreMesh(core_axis_name="c", subcore_axis_name="s"),
        compiler_params=pltpu.CompilerParams(needs_layout_passes=False),
        scratch_types=[
            pltpu.VMEM((N + 128,), jnp.float32),
            pltpu.VMEM((N + 128,), jnp.float32),
            pltpu.VMEM((N + 128,), jnp.float32),
            pltpu.VMEM((16,), jnp.float32),
            pltpu.VMEM((8, W), jnp.int32),
            pltpu.VMEM((8, W), jnp.int32),
            pltpu.VMEM((8, W), jnp.float32),
            pltpu.VMEM((8, W), jnp.float32),
            pltpu.VMEM((8, W), jnp.float32),
            pltpu.VMEM((8, W), jnp.float32),
            pltpu.VMEM((8, W), jnp.float32),
            pltpu.VMEM((8, W), jnp.float32),
            pltpu.VMEM((8, W), jnp.float32),
            pltpu.VMEM((8, W), jnp.float32),
            pltpu.VMEM((8, W), jnp.float32),
            pltpu.VMEM((8, W), jnp.float32),
            pltpu.SemaphoreType.DMA,
            pltpu.SemaphoreType.DMA,
            pltpu.SemaphoreType.DMA,
            pltpu.SemaphoreType.DMA,
        ],
    )(pos_1d, nbr_t, mask_t, cell_1d, off_t)
    return jnp.transpose(out.reshape(B, K, N), (0, 2, 1))
